# baseline (speedup 1.0000x reference)
import functools

import jax
import jax.numpy as jnp
from jax import lax
from jax.experimental import pallas as pl
from jax.experimental.pallas import tpu as pltpu
from jax.experimental.pallas import tpu_sc as plsc

_info = plsc.get_sparse_core_info()
_NC, _NS = _info.num_cores, _info.num_subcores
_NW = _NC * _NS

_CHUNK = 32


def _make_gather_mean(n_batch, d_feat, n_sample):
    mesh = plsc.VectorSubcoreMesh(core_axis_name="c", subcore_axis_name="s")
    n_nb = n_batch * n_sample
    per_b = n_batch // _NW
    nd = d_feat // 16
    ch = _CHUNK
    nsch = n_sample * ch
    nch_b = per_b // ch
    nch = nch_b * (1 + n_sample)

    @functools.partial(
        pl.kernel,
        mesh=mesh,
        out_type=[
            jax.ShapeDtypeStruct((n_batch, d_feat), jnp.float32),
            jax.ShapeDtypeStruct((n_batch, d_feat), jnp.float32),
            jax.ShapeDtypeStruct((n_sample, n_batch, d_feat), jnp.float32),
            jax.ShapeDtypeStruct((n_sample, n_batch, d_feat), jnp.float32),
        ],
        scratch_types=[
            pltpu.VMEM(((1 + n_sample) * per_b,), jnp.int32),
            pltpu.VMEM((nsch,), jnp.int32),
            pltpu.VMEM((nsch,), jnp.int32),
            pltpu.VMEM((nsch,), jnp.int32),
            pltpu.VMEM((nsch,), jnp.int32),
            pltpu.VMEM((2 * ch, d_feat), jnp.float32),
            pltpu.VMEM((2 * nsch, d_feat), jnp.float32),
            pltpu.VMEM((ch, d_feat), jnp.float32),
            pltpu.SemaphoreType.DMA,
            pltpu.SemaphoreType.DMA,
            pltpu.SemaphoreType.DMA,
        ],
    )
    def k(nodes_b_hbm, feats_hbm, neighf_hbm,
          sb_out, ab_out, sn_out, an_out,
          allnodes_v, eidx_v0, eidx_v1, nids_v0, nids_v1,
          self_v, nb_v, agg_v, semi, semf0, semf1):
        w = lax.axis_index("s") * _NC + lax.axis_index("c")
        semf = (semf0, semf1)
        eidx = (eidx_v0, eidx_v1)
        nids = (nids_v0, nids_v1)

        def off_in(g):
            q = jnp.maximum(g - nch_b, 0)
            s2 = q // nch_b
            g2 = q - s2 * nch_b
            return jnp.where(g < nch_b, g * ch,
                             per_b + s2 * per_b + g2 * ch)


        def launch_ids(g, slot):
            src = off_in(g)
            for s in range(n_sample):
                for grp in range(ch // 16):
                    v = allnodes_v[pl.ds(src + grp * 16, 16)]
                    eidx[slot][pl.ds(s * ch + grp * 16, 16)] = (
                        v * n_sample + s)
            pltpu.async_copy(neighf_hbm.at[eidx[slot]], nids[slot], semi)

        def launch_feats(g, fslot):
            pltpu.make_async_copy(neighf_hbm.at[pl.ds(0, nsch)],
                                  nids[fslot], semi).wait()
            pltpu.async_copy(
                feats_hbm.at[allnodes_v.at[pl.ds(off_in(g), ch)]],
                self_v.at[pl.ds(fslot * ch, ch)], semf[fslot])
            pltpu.async_copy(feats_hbm.at[nids[fslot]],
                             nb_v.at[pl.ds(fslot * nsch, nsch)], semf[fslot])

        def wait_and_stash(g, fslot):
            pltpu.make_async_copy(feats_hbm.at[pl.ds(0, ch)],
                                  self_v.at[pl.ds(fslot * ch, ch)],
                                  semf[fslot]).wait()
            pltpu.make_async_copy(feats_hbm.at[pl.ds(0, nsch)],
                                  nb_v.at[pl.ds(fslot * nsch, nsch)],
                                  semf[fslot]).wait()

            @pl.when(g < nch_b)
            def _():
                for s in range(n_sample):
                    for grp in range(ch // 16):
                        allnodes_v[pl.ds(per_b + s * per_b + g * ch
                                         + grp * 16, 16)] = (
                            nids[fslot][pl.ds(s * ch + grp * 16, 16)])

        def finish(g, fslot):
            is_p1 = g < nch_b
            base = fslot * nsch

            def node_body(c, _):
                for d in range(nd):
                    sl = pl.ds(d * 16, 16)
                    acc = nb_v[base + c, sl]
                    for s in range(1, n_sample):
                        acc = acc + nb_v[base + s * ch + c, sl]
                    agg_v[c, sl] = acc * jnp.float32(1.0 / n_sample)
                return 0

            lax.fori_loop(0, ch, node_body, 0)
            q = jnp.maximum(g - nch_b, 0)
            s2 = q // nch_b
            g2 = q - s2 * nch_b
            row2 = w * per_b + g2 * ch
            sv = self_v.at[pl.ds(fslot * ch, ch)]

            @pl.when(is_p1)
            def _():
                oo = w * per_b + g * ch
                pltpu.sync_copy(sv, sb_out.at[pl.ds(oo, ch)])
                pltpu.sync_copy(agg_v, ab_out.at[pl.ds(oo, ch)])

            @pl.when(jnp.logical_not(is_p1))
            def _():
                pltpu.sync_copy(sv, sn_out.at[s2, pl.ds(row2, ch)])
                pltpu.sync_copy(agg_v, an_out.at[s2, pl.ds(row2, ch)])

        pltpu.sync_copy(nodes_b_hbm.at[pl.ds(w * per_b, per_b)],
                        allnodes_v.at[pl.ds(0, per_b)])
        launch_ids(0, 0)
        launch_feats(0, 0)
        launch_ids(1, 1)

        def iteration(g, slot):
            @pl.when(g + 1 < nch)
            def _():
                launch_feats(g + 1, 1 - slot)

            wait_and_stash(g, slot)

            @pl.when(g + 2 < nch)
            def _():
                launch_ids(g + 2, slot)

            finish(g, slot)

        def pair_body(p, _):
            iteration(2 * p, 0)
            iteration(2 * p + 1, 1)
            return 0

        lax.fori_loop(0, nch // 2, pair_body, 0)

    return k


def _tc_body(n_sample, r_blk, d_out, d_feat,
             sb_ref, ab_ref, sn_ref, an_ref, w1_ref, w2_ref, o_ref):
    f32 = jnp.float32

    def dott(x, wh):
        return lax.dot_general(x, wh, (((1,), (1,)), ((), ())),
                               preferred_element_type=f32)

    w1a = w1_ref[:, :d_feat]
    w1b = w1_ref[:, d_feat:]
    h_self = jax.nn.relu(dott(sb_ref[...], w1a) + dott(ab_ref[...], w1b))
    sn = sn_ref[...].reshape(n_sample * r_blk, -1)
    an = an_ref[...].reshape(n_sample * r_blk, -1)
    h_nb = jax.nn.relu(dott(sn, w1a) + dott(an, w1b))
    agg2 = jnp.mean(h_nb.reshape(n_sample, r_blk, d_out), axis=0)
    o_ref[...] = jax.nn.relu(dott(h_self, w2_ref[:, :d_out]) +
                             dott(agg2, w2_ref[:, d_out:]))


_SPLIT = 2


def kernel(nodes_batch, feats, neigh, W1, W2):
    n_batch, = nodes_batch.shape
    n_nodes, d_feat = feats.shape
    n_sample = neigh.shape[1]
    d_out = W1.shape[0]

    neighf = neigh.reshape(-1)
    nh = n_batch // _SPLIT
    sc = _make_gather_mean(nh, d_feat, n_sample)
    r_blk = 256
    grid = (nh // r_blk,)
    wspec = pl.BlockSpec((d_out, 2 * d_feat), lambda i: (0, 0))
    tc = pl.pallas_call(
        functools.partial(_tc_body, n_sample, r_blk, d_out, d_feat),
        grid=grid,
        in_specs=[
            pl.BlockSpec((r_blk, d_feat), lambda i: (i, 0)),
            pl.BlockSpec((r_blk, d_feat), lambda i: (i, 0)),
            pl.BlockSpec((n_sample, r_blk, d_feat), lambda i: (0, i, 0)),
            pl.BlockSpec((n_sample, r_blk, d_feat), lambda i: (0, i, 0)),
            wspec, wspec,
        ],
        out_specs=pl.BlockSpec((r_blk, d_out), lambda i: (i, 0)),
        out_shape=jax.ShapeDtypeStruct((nh, d_out), jnp.float32),
    )
    parts = []
    for h in range(_SPLIT):
        sb, ab, sn3, an3 = sc(
            lax.slice_in_dim(nodes_batch, h * nh, (h + 1) * nh), feats,
            neighf)
        parts.append(tc(sb, ab, sn3, an3, W1, W2))
    return jnp.concatenate(parts, axis=0)


# speedup vs baseline: 1.0284x; 1.0284x over previous
import functools

import jax
import jax.numpy as jnp
from jax import lax
from jax.experimental import pallas as pl
from jax.experimental.pallas import tpu as pltpu
from jax.experimental.pallas import tpu_sc as plsc

_info = plsc.get_sparse_core_info()
_NC, _NS = _info.num_cores, _info.num_subcores
_NW = _NC * _NS

_CHUNK = 32


def _make_gather_mean(n_batch, d_feat, n_sample):
    mesh = plsc.VectorSubcoreMesh(core_axis_name="c", subcore_axis_name="s")
    n_nb = n_batch * n_sample
    per_b = n_batch // _NW
    nd = d_feat // 16
    ch = _CHUNK
    nsch = n_sample * ch
    nch_b = per_b // ch
    nch = nch_b * (1 + n_sample)

    @functools.partial(
        pl.kernel,
        mesh=mesh,
        out_type=[
            jax.ShapeDtypeStruct((n_batch, d_feat), jnp.float32),
            jax.ShapeDtypeStruct((n_batch, d_feat), jnp.float32),
            jax.ShapeDtypeStruct((n_sample, n_batch, d_feat), jnp.float32),
            jax.ShapeDtypeStruct((n_sample, n_batch, d_feat), jnp.float32),
        ],
        scratch_types=[
            pltpu.VMEM(((1 + n_sample) * per_b,), jnp.int32),
            pltpu.VMEM((nsch,), jnp.int32),
            pltpu.VMEM((nsch,), jnp.int32),
            pltpu.VMEM((nsch,), jnp.int32),
            pltpu.VMEM((nsch,), jnp.int32),
            pltpu.VMEM((2 * ch, d_feat), jnp.float32),
            pltpu.VMEM((2 * nsch, d_feat), jnp.float32),
            pltpu.VMEM((2 * ch, d_feat), jnp.float32),
            pltpu.SemaphoreType.DMA,
            pltpu.SemaphoreType.DMA,
            pltpu.SemaphoreType.DMA,
            pltpu.SemaphoreType.DMA,
            pltpu.SemaphoreType.DMA,
        ],
    )
    def k(nodes_b_hbm, feats_hbm, neighf_hbm,
          sb_out, ab_out, sn_out, an_out,
          allnodes_v, eidx_v0, eidx_v1, nids_v0, nids_v1,
          self_v, nb_v, agg_v, semi, semf0, semf1, semo0, semo1):
        w = lax.axis_index("s") * _NC + lax.axis_index("c")
        semf = (semf0, semf1)
        semo = (semo0, semo1)
        eidx = (eidx_v0, eidx_v1)
        nids = (nids_v0, nids_v1)

        def off_in(g):
            q = jnp.maximum(g - nch_b, 0)
            s2 = q // nch_b
            g2 = q - s2 * nch_b
            return jnp.where(g < nch_b, g * ch,
                             per_b + s2 * per_b + g2 * ch)


        def launch_ids(g, slot):
            src = off_in(g)
            for s in range(n_sample):
                for grp in range(ch // 16):
                    v = allnodes_v[pl.ds(src + grp * 16, 16)]
                    eidx[slot][pl.ds(s * ch + grp * 16, 16)] = (
                        v * n_sample + s)
            pltpu.async_copy(neighf_hbm.at[eidx[slot]], nids[slot], semi)

        def launch_feats(g, fslot):
            g = jnp.int32(g)
            pltpu.make_async_copy(neighf_hbm.at[pl.ds(0, nsch)],
                                  nids[fslot], semi).wait()

            @pl.when(g >= 2)
            def _():
                pltpu.make_async_copy(
                    self_v.at[pl.ds(fslot * ch, ch)],
                    sb_out.at[pl.ds(0, ch)], semo[fslot]).wait()
                pltpu.make_async_copy(
                    agg_v.at[pl.ds(fslot * ch, ch)],
                    ab_out.at[pl.ds(0, ch)], semo[fslot]).wait()
            pltpu.async_copy(
                feats_hbm.at[allnodes_v.at[pl.ds(off_in(g), ch)]],
                self_v.at[pl.ds(fslot * ch, ch)], semf[fslot])
            pltpu.async_copy(feats_hbm.at[nids[fslot]],
                             nb_v.at[pl.ds(fslot * nsch, nsch)], semf[fslot])

        def wait_and_stash(g, fslot):
            pltpu.make_async_copy(feats_hbm.at[pl.ds(0, ch)],
                                  self_v.at[pl.ds(fslot * ch, ch)],
                                  semf[fslot]).wait()
            pltpu.make_async_copy(feats_hbm.at[pl.ds(0, nsch)],
                                  nb_v.at[pl.ds(fslot * nsch, nsch)],
                                  semf[fslot]).wait()

            @pl.when(g < nch_b)
            def _():
                for s in range(n_sample):
                    for grp in range(ch // 16):
                        allnodes_v[pl.ds(per_b + s * per_b + g * ch
                                         + grp * 16, 16)] = (
                            nids[fslot][pl.ds(s * ch + grp * 16, 16)])

        def finish(g, fslot):
            is_p1 = g < nch_b
            base = fslot * nsch

            def node_body(c, _):
                for d in range(nd):
                    sl = pl.ds(d * 16, 16)
                    acc = nb_v[base + c, sl]
                    for s in range(1, n_sample):
                        acc = acc + nb_v[base + s * ch + c, sl]
                    agg_v[fslot * ch + c, sl] = acc * jnp.float32(1.0 / n_sample)
                return 0

            lax.fori_loop(0, ch, node_body, 0)
            q = jnp.maximum(g - nch_b, 0)
            s2 = q // nch_b
            g2 = q - s2 * nch_b
            row2 = w * per_b + g2 * ch
            sv = self_v.at[pl.ds(fslot * ch, ch)]
            av = agg_v.at[pl.ds(fslot * ch, ch)]

            @pl.when(is_p1)
            def _():
                oo = w * per_b + g * ch
                pltpu.async_copy(sv, sb_out.at[pl.ds(oo, ch)], semo[fslot])
                pltpu.async_copy(av, ab_out.at[pl.ds(oo, ch)], semo[fslot])

            @pl.when(jnp.logical_not(is_p1))
            def _():
                pltpu.async_copy(sv, sn_out.at[s2, pl.ds(row2, ch)],
                                 semo[fslot])
                pltpu.async_copy(av, an_out.at[s2, pl.ds(row2, ch)],
                                 semo[fslot])

        pltpu.sync_copy(nodes_b_hbm.at[pl.ds(w * per_b, per_b)],
                        allnodes_v.at[pl.ds(0, per_b)])
        launch_ids(0, 0)
        launch_feats(0, 0)
        launch_ids(1, 1)

        def iteration(g, slot):
            @pl.when(g + 1 < nch)
            def _():
                launch_feats(g + 1, 1 - slot)

            wait_and_stash(g, slot)

            @pl.when(g + 2 < nch)
            def _():
                launch_ids(g + 2, slot)

            finish(g, slot)

        def pair_body(p, _):
            iteration(2 * p, 0)
            iteration(2 * p + 1, 1)
            return 0

        lax.fori_loop(0, nch // 2, pair_body, 0)
        for fslot in range(2):
            pltpu.make_async_copy(self_v.at[pl.ds(fslot * ch, ch)],
                                  sb_out.at[pl.ds(0, ch)],
                                  semo[fslot]).wait()
            pltpu.make_async_copy(agg_v.at[pl.ds(fslot * ch, ch)],
                                  ab_out.at[pl.ds(0, ch)],
                                  semo[fslot]).wait()

    return k


def _tc_body(n_sample, r_blk, d_out, d_feat,
             sb_ref, ab_ref, sn_ref, an_ref, w1_ref, w2_ref, o_ref):
    f32 = jnp.float32

    def dott(x, wh):
        return lax.dot_general(x, wh, (((1,), (1,)), ((), ())),
                               preferred_element_type=f32)

    w1a = w1_ref[:, :d_feat]
    w1b = w1_ref[:, d_feat:]
    h_self = jax.nn.relu(dott(sb_ref[...], w1a) + dott(ab_ref[...], w1b))
    sn = sn_ref[...].reshape(n_sample * r_blk, -1)
    an = an_ref[...].reshape(n_sample * r_blk, -1)
    h_nb = jax.nn.relu(dott(sn, w1a) + dott(an, w1b))
    agg2 = jnp.mean(h_nb.reshape(n_sample, r_blk, d_out), axis=0)
    o_ref[...] = jax.nn.relu(dott(h_self, w2_ref[:, :d_out]) +
                             dott(agg2, w2_ref[:, d_out:]))


_SPLIT = 1


def kernel(nodes_batch, feats, neigh, W1, W2):
    n_batch, = nodes_batch.shape
    n_nodes, d_feat = feats.shape
    n_sample = neigh.shape[1]
    d_out = W1.shape[0]

    neighf = neigh.reshape(-1)
    nh = n_batch // _SPLIT
    sc = _make_gather_mean(nh, d_feat, n_sample)
    r_blk = 256
    grid = (nh // r_blk,)
    wspec = pl.BlockSpec((d_out, 2 * d_feat), lambda i: (0, 0))
    tc = pl.pallas_call(
        functools.partial(_tc_body, n_sample, r_blk, d_out, d_feat),
        grid=grid,
        in_specs=[
            pl.BlockSpec((r_blk, d_feat), lambda i: (i, 0)),
            pl.BlockSpec((r_blk, d_feat), lambda i: (i, 0)),
            pl.BlockSpec((n_sample, r_blk, d_feat), lambda i: (0, i, 0)),
            pl.BlockSpec((n_sample, r_blk, d_feat), lambda i: (0, i, 0)),
            wspec, wspec,
        ],
        out_specs=pl.BlockSpec((r_blk, d_out), lambda i: (i, 0)),
        out_shape=jax.ShapeDtypeStruct((nh, d_out), jnp.float32),
    )
    parts = []
    for h in range(_SPLIT):
        sb, ab, sn3, an3 = sc(
            lax.slice_in_dim(nodes_batch, h * nh, (h + 1) * nh), feats,
            neighf)
        parts.append(tc(sb, ab, sn3, an3, W1, W2))
    return jnp.concatenate(parts, axis=0)


# speedup vs baseline: 1.4537x; 1.4135x over previous
import functools

import jax
import jax.numpy as jnp
from jax import lax
from jax.experimental import pallas as pl
from jax.experimental.pallas import tpu as pltpu
from jax.experimental.pallas import tpu_sc as plsc

_info = plsc.get_sparse_core_info()
_NC, _NS = _info.num_cores, _info.num_subcores
_NW = _NC * _NS

_CHUNK = 32


def _make_gather_mean(n_batch, d_feat, n_sample, n_nodes):
    mesh = plsc.VectorSubcoreMesh(core_axis_name="c", subcore_axis_name="s")
    n_nb = n_batch * n_sample
    per_b = n_batch // _NW
    nd = d_feat // 16
    ch = _CHUNK
    nsch = n_sample * ch
    nch_b = per_b // ch
    nch = nch_b * (1 + n_sample)

    @functools.partial(
        pl.kernel,
        mesh=mesh,
        out_type=[
            jax.ShapeDtypeStruct((n_batch, d_feat), jnp.float32),
            jax.ShapeDtypeStruct((n_batch, d_feat), jnp.float32),
            jax.ShapeDtypeStruct((n_sample, n_batch, d_feat), jnp.float32),
            jax.ShapeDtypeStruct((n_sample, n_batch, d_feat), jnp.float32),
        ],
        scratch_types=[
            pltpu.VMEM(((1 + n_sample) * per_b,), jnp.int32),
            pltpu.VMEM((nsch,), jnp.int32),
            pltpu.VMEM((nsch,), jnp.int32),
            pltpu.VMEM((nsch,), jnp.int32),
            pltpu.VMEM((nsch,), jnp.int32),
            pltpu.VMEM((2 * ch, d_feat), jnp.float32),
            pltpu.VMEM((2 * nsch, d_feat), jnp.float32),
            pltpu.VMEM((2 * ch, d_feat), jnp.float32),
            pltpu.SemaphoreType.DMA,
            pltpu.SemaphoreType.DMA,
            pltpu.SemaphoreType.DMA,
            pltpu.SemaphoreType.DMA,
            pltpu.SemaphoreType.DMA,
        ],
    )
    def k(nodes_b_hbm, feats_hbm, neighf_hbm,
          sb_out, ab_out, sn_out, an_out,
          allnodes_v, eidx_v0, eidx_v1, nids_v0, nids_v1,
          self_v, nb_v, agg_v, semi, semf0, semf1, semo0, semo1):
        w = lax.axis_index("s") * _NC + lax.axis_index("c")
        semf = (semf0, semf1)
        semo = (semo0, semo1)
        eidx = (eidx_v0, eidx_v1)
        nids = (nids_v0, nids_v1)

        def off_in(g):
            q = jnp.maximum(g - nch_b, 0)
            s2 = q // nch_b
            g2 = q - s2 * nch_b
            return jnp.where(g < nch_b, g * ch,
                             per_b + s2 * per_b + g2 * ch)


        def launch_ids(g, slot):
            src = off_in(g)
            for s in range(n_sample):
                for grp in range(ch // 16):
                    v = allnodes_v[pl.ds(src + grp * 16, 16)]
                    eidx[slot][pl.ds(s * ch + grp * 16, 16)] = (
                        v + s * n_nodes)
            pltpu.async_copy(neighf_hbm.at[eidx[slot]], nids[slot], semi)

        def launch_feats(g, fslot):
            g = jnp.int32(g)
            pltpu.make_async_copy(neighf_hbm.at[pl.ds(0, nsch)],
                                  nids[fslot], semi).wait()

            @pl.when(g >= 2)
            def _():
                pltpu.make_async_copy(
                    self_v.at[pl.ds(fslot * ch, ch)],
                    sb_out.at[pl.ds(0, ch)], semo[fslot]).wait()
                pltpu.make_async_copy(
                    agg_v.at[pl.ds(fslot * ch, ch)],
                    ab_out.at[pl.ds(0, ch)], semo[fslot]).wait()
            pltpu.async_copy(
                feats_hbm.at[allnodes_v.at[pl.ds(off_in(g), ch)]],
                self_v.at[pl.ds(fslot * ch, ch)], semf[fslot])
            pltpu.async_copy(feats_hbm.at[nids[fslot]],
                             nb_v.at[pl.ds(fslot * nsch, nsch)], semf[fslot])

        def wait_and_stash(g, fslot):
            pltpu.make_async_copy(feats_hbm.at[pl.ds(0, ch)],
                                  self_v.at[pl.ds(fslot * ch, ch)],
                                  semf[fslot]).wait()
            pltpu.make_async_copy(feats_hbm.at[pl.ds(0, nsch)],
                                  nb_v.at[pl.ds(fslot * nsch, nsch)],
                                  semf[fslot]).wait()

            @pl.when(g < nch_b)
            def _():
                for s in range(n_sample):
                    for grp in range(ch // 16):
                        allnodes_v[pl.ds(per_b + s * per_b + g * ch
                                         + grp * 16, 16)] = (
                            nids[fslot][pl.ds(s * ch + grp * 16, 16)])

        def finish(g, fslot):
            is_p1 = g < nch_b
            base = fslot * nsch

            def node_body(c, _):
                for d in range(nd):
                    sl = pl.ds(d * 16, 16)
                    acc = nb_v[base + c, sl]
                    for s in range(1, n_sample):
                        acc = acc + nb_v[base + s * ch + c, sl]
                    agg_v[fslot * ch + c, sl] = acc * jnp.float32(1.0 / n_sample)
                return 0

            lax.fori_loop(0, ch, node_body, 0)
            q = jnp.maximum(g - nch_b, 0)
            s2 = q // nch_b
            g2 = q - s2 * nch_b
            row2 = w * per_b + g2 * ch
            sv = self_v.at[pl.ds(fslot * ch, ch)]
            av = agg_v.at[pl.ds(fslot * ch, ch)]

            @pl.when(is_p1)
            def _():
                oo = w * per_b + g * ch
                pltpu.async_copy(sv, sb_out.at[pl.ds(oo, ch)], semo[fslot])
                pltpu.async_copy(av, ab_out.at[pl.ds(oo, ch)], semo[fslot])

            @pl.when(jnp.logical_not(is_p1))
            def _():
                pltpu.async_copy(sv, sn_out.at[s2, pl.ds(row2, ch)],
                                 semo[fslot])
                pltpu.async_copy(av, an_out.at[s2, pl.ds(row2, ch)],
                                 semo[fslot])

        pltpu.sync_copy(nodes_b_hbm.at[pl.ds(w * per_b, per_b)],
                        allnodes_v.at[pl.ds(0, per_b)])
        launch_ids(0, 0)
        launch_feats(0, 0)
        launch_ids(1, 1)

        def iteration(g, slot):
            @pl.when(g + 1 < nch)
            def _():
                launch_feats(g + 1, 1 - slot)

            wait_and_stash(g, slot)

            @pl.when(g + 2 < nch)
            def _():
                launch_ids(g + 2, slot)

            finish(g, slot)

        def pair_body(p, _):
            iteration(2 * p, 0)
            iteration(2 * p + 1, 1)
            return 0

        lax.fori_loop(0, nch // 2, pair_body, 0)
        for fslot in range(2):
            pltpu.make_async_copy(self_v.at[pl.ds(fslot * ch, ch)],
                                  sb_out.at[pl.ds(0, ch)],
                                  semo[fslot]).wait()
            pltpu.make_async_copy(agg_v.at[pl.ds(fslot * ch, ch)],
                                  ab_out.at[pl.ds(0, ch)],
                                  semo[fslot]).wait()

    return k


def _tc_body(n_sample, r_blk, d_out, d_feat,
             sb_ref, ab_ref, sn_ref, an_ref, w1_ref, w2_ref, o_ref):
    f32 = jnp.float32

    def dott(x, wh):
        return lax.dot_general(x, wh, (((1,), (1,)), ((), ())),
                               preferred_element_type=f32)

    w1a = w1_ref[:, :d_feat]
    w1b = w1_ref[:, d_feat:]
    h_self = jax.nn.relu(dott(sb_ref[...], w1a) + dott(ab_ref[...], w1b))
    sn = sn_ref[...].reshape(n_sample * r_blk, -1)
    an = an_ref[...].reshape(n_sample * r_blk, -1)
    h_nb = jax.nn.relu(dott(sn, w1a) + dott(an, w1b))
    agg2 = jnp.mean(h_nb.reshape(n_sample, r_blk, d_out), axis=0)
    o_ref[...] = jax.nn.relu(dott(h_self, w2_ref[:, :d_out]) +
                             dott(agg2, w2_ref[:, d_out:]))


_SPLIT = 1


def kernel(nodes_batch, feats, neigh, W1, W2):
    n_batch, = nodes_batch.shape
    n_nodes, d_feat = feats.shape
    n_sample = neigh.shape[1]
    d_out = W1.shape[0]

    neighf = jnp.swapaxes(neigh, 0, 1).reshape(-1)
    nh = n_batch // _SPLIT
    sc = _make_gather_mean(nh, d_feat, n_sample, n_nodes)
    r_blk = 256
    grid = (nh // r_blk,)
    wspec = pl.BlockSpec((d_out, 2 * d_feat), lambda i: (0, 0))
    tc = pl.pallas_call(
        functools.partial(_tc_body, n_sample, r_blk, d_out, d_feat),
        grid=grid,
        in_specs=[
            pl.BlockSpec((r_blk, d_feat), lambda i: (i, 0)),
            pl.BlockSpec((r_blk, d_feat), lambda i: (i, 0)),
            pl.BlockSpec((n_sample, r_blk, d_feat), lambda i: (0, i, 0)),
            pl.BlockSpec((n_sample, r_blk, d_feat), lambda i: (0, i, 0)),
            wspec, wspec,
        ],
        out_specs=pl.BlockSpec((r_blk, d_out), lambda i: (i, 0)),
        out_shape=jax.ShapeDtypeStruct((nh, d_out), jnp.float32),
    )
    parts = []
    for h in range(_SPLIT):
        sb, ab, sn3, an3 = sc(
            lax.slice_in_dim(nodes_batch, h * nh, (h + 1) * nh), feats,
            neighf)
        parts.append(tc(sb, ab, sn3, an3, W1, W2))
    return jnp.concatenate(parts, axis=0)
